# baseline (speedup 1.0000x reference)
import functools

import jax
import jax.numpy as jnp
from jax import lax
from jax.experimental import pallas as pl
from jax.experimental.pallas import tpu as pltpu
from jax.experimental.pallas import tpu_sc as plsc

NUM_CORES = 2
NUM_SUBCORES = 16
LANES = 16
NW = NUM_CORES * NUM_SUBCORES

BATCH = 16384
LATENT = 16
B_PER_W = BATCH // NW
CHUNK = 128
NCHUNK = B_PER_W // CHUNK


def _body(users_ref, items_ref, utab_ref, itab_ref, out_ref,
          idx_u, idx_i, rows_u, rows_i, out_v, sem):
    wid = lax.axis_index("s") * NUM_CORES + lax.axis_index("c")
    base = wid * B_PER_W

    pltpu.sync_copy(users_ref.at[wid], idx_u)
    pltpu.sync_copy(items_ref.at[wid], idx_i)

    copies = []
    for j in range(NCHUNK):
        copies.append(pltpu.make_async_copy(
            utab_ref.at[idx_u.at[j]], rows_u.at[pl.ds(j * CHUNK, CHUNK)], sem))
        copies.append(pltpu.make_async_copy(
            itab_ref.at[idx_i.at[j]], rows_i.at[pl.ds(j * CHUNK, CHUNK)], sem))
    for c in copies:
        c.start()
    for c in copies:
        c.wait()

    iota = lax.iota(jnp.int32, LANES)

    def group(g, _):
        row_idx = g * LANES + iota
        acc = jnp.zeros((LANES,), jnp.float32)
        for d in range(LATENT):
            col = jnp.full((LANES,), d, jnp.int32)
            u = plsc.load_gather(rows_u, [row_idx, col])
            v = plsc.load_gather(rows_i, [row_idx, col])
            acc = acc + u * v
        y = 1.0 / (1.0 + jnp.exp(-acc))
        out_v[pl.ds(g * LANES, LANES)] = y
        return 0

    lax.fori_loop(0, B_PER_W // LANES, group, 0)

    pltpu.sync_copy(out_v, out_ref.at[pl.ds(base, B_PER_W)])


@jax.jit
def kernel(users, items, user_table, item_table):
    users3 = users.reshape(NW, NCHUNK, CHUNK)
    items3 = items.reshape(NW, NCHUNK, CHUNK)
    mesh = plsc.VectorSubcoreMesh(
        core_axis_name="c", subcore_axis_name="s",
        num_cores=NUM_CORES, num_subcores=NUM_SUBCORES)
    run = pl.kernel(
        _body,
        out_type=jax.ShapeDtypeStruct((BATCH,), jnp.float32),
        mesh=mesh,
        scratch_types=[
            pltpu.VMEM((NCHUNK, CHUNK), jnp.int32),
            pltpu.VMEM((NCHUNK, CHUNK), jnp.int32),
            pltpu.VMEM((B_PER_W, LATENT), jnp.float32),
            pltpu.VMEM((B_PER_W, LATENT), jnp.float32),
            pltpu.VMEM((B_PER_W,), jnp.float32),
            pltpu.SemaphoreType.DMA,
        ],
        compiler_params=pltpu.CompilerParams(
            needs_layout_passes=False, use_tc_tiling_on_sc=False),
    )
    return run(users3, items3, user_table, item_table)
